# baseline (speedup 1.0000x reference)
import jax
import jax.numpy as jnp
from jax.experimental import pallas as pl
from jax.experimental.pallas import tpu as pltpu

N_EXPERTS = 8
TOPK = 2
TILE = 1024
DCH = 128
CPT = TILE // DCH
NTBUF = 2


def _route_tile(xc, w, col0, vals_ref, idx_ref):
    s = jax.lax.dot_general(
        w, xc,
        dimension_numbers=(((1,), (1,)), ((), ())),
        preferred_element_type=jnp.float32,
        precision=jax.lax.Precision.DEFAULT,
    )
    subl = jax.lax.broadcasted_iota(jnp.int32, s.shape, 0).astype(jnp.float32)
    neg_big = jnp.float32(-3.0e38)
    many = jnp.float32(N_EXPERTS)
    m1 = jnp.max(s, axis=0, keepdims=True)
    i1 = jnp.min(jnp.where(s == m1, subl, many), axis=0, keepdims=True)
    s2 = jnp.where(subl == i1, neg_big, s)
    m2 = jnp.max(s2, axis=0, keepdims=True)
    i2 = jnp.min(jnp.where(s2 == m2, subl, many), axis=0, keepdims=True)
    e2 = jnp.exp(m2 - m1)
    denom = 1.0 + e2
    vals_ref[:, pl.ds(col0, TILE)] = jnp.concatenate([1.0 / denom, e2 / denom], axis=0)
    idx_ref[:, pl.ds(col0, TILE)] = jnp.concatenate([i1, i2], axis=0).astype(jnp.int32)


def _router_body(x_hbm, w_ref, vals_ref, idx_ref, buf, sems):
    n_tiles = x_hbm.shape[0] // TILE
    w = w_ref[...]

    def copy(t, tb, c):
        return pltpu.make_async_copy(
            x_hbm.at[pl.ds(t * TILE + c * DCH, DCH), :],
            buf.at[tb, pl.ds(c * DCH, DCH), :],
            sems.at[tb, c],
        )

    for t in range(min(NTBUF, n_tiles)):
        for c in range(CPT):
            copy(t, t, c).start()
    for t in range(n_tiles):
        tb = t % NTBUF
        for c in range(CPT):
            copy(t, tb, c).wait()
        _route_tile(buf[tb], w, t * TILE, vals_ref, idx_ref)
        if t + NTBUF < n_tiles:
            for c in range(CPT):
                copy(t + NTBUF, tb, c).start()


def kernel(x, W_router):
    batch, seqlen, hidden = x.shape
    n_tokens = batch * seqlen
    x_flat = x.reshape(n_tokens, hidden)
    vals_t, idx_t = pl.pallas_call(
        _router_body,
        in_specs=[
            pl.BlockSpec(memory_space=pl.ANY),
            pl.BlockSpec(memory_space=pltpu.VMEM),
        ],
        out_specs=[
            pl.BlockSpec(memory_space=pltpu.VMEM),
            pl.BlockSpec(memory_space=pltpu.VMEM),
        ],
        out_shape=[
            jax.ShapeDtypeStruct((TOPK, n_tokens), jnp.float32),
            jax.ShapeDtypeStruct((TOPK, n_tokens), jnp.int32),
        ],
        scratch_shapes=[
            pltpu.VMEM((NTBUF, TILE, hidden), jnp.float32),
            pltpu.SemaphoreType.DMA((NTBUF, CPT)),
        ],
    )(x_flat, W_router)
    return (vals_t.T, idx_t.T)


# speedup vs baseline: 1.0520x; 1.0520x over previous
import jax
import jax.numpy as jnp
from jax.experimental import pallas as pl
from jax.experimental.pallas import tpu as pltpu

N_EXPERTS = 8
TOPK = 2
TILE = 512
DCH = 256
CPT = TILE // DCH
NTBUF = 2


def _route_tile(xc, w, col0, vals_ref, idx_ref):
    s = jax.lax.dot_general(
        w, xc,
        dimension_numbers=(((1,), (1,)), ((), ())),
        preferred_element_type=jnp.float32,
        precision=jax.lax.Precision.DEFAULT,
    )
    subl = jax.lax.broadcasted_iota(jnp.int32, s.shape, 0).astype(jnp.float32)
    neg_big = jnp.float32(-3.0e38)
    many = jnp.float32(N_EXPERTS)
    m1 = jnp.max(s, axis=0, keepdims=True)
    i1 = jnp.min(jnp.where(s == m1, subl, many), axis=0, keepdims=True)
    s2 = jnp.where(subl == i1, neg_big, s)
    m2 = jnp.max(s2, axis=0, keepdims=True)
    i2 = jnp.min(jnp.where(s2 == m2, subl, many), axis=0, keepdims=True)
    e2 = jnp.exp(m2 - m1)
    denom = 1.0 + e2
    vals_ref[:, pl.ds(col0, TILE)] = jnp.concatenate([1.0 / denom, e2 / denom], axis=0)
    idx_ref[:, pl.ds(col0, TILE)] = jnp.concatenate([i1, i2], axis=0).astype(jnp.int32)


def _router_body(x_hbm, w_ref, vals_ref, idx_ref, buf, sems):
    n_tiles = x_hbm.shape[0] // TILE
    w = w_ref[...]

    def copy(t, tb, c):
        return pltpu.make_async_copy(
            x_hbm.at[pl.ds(t * TILE + c * DCH, DCH), :],
            buf.at[tb, pl.ds(c * DCH, DCH), :],
            sems.at[tb, c],
        )

    for t in range(min(NTBUF, n_tiles)):
        for c in range(CPT):
            copy(t, t, c).start()
    for t in range(n_tiles):
        tb = t % NTBUF
        for c in range(CPT):
            copy(t, tb, c).wait()
        _route_tile(buf[tb], w, t * TILE, vals_ref, idx_ref)
        if t + NTBUF < n_tiles:
            for c in range(CPT):
                copy(t + NTBUF, tb, c).start()


def kernel(x, W_router):
    batch, seqlen, hidden = x.shape
    n_tokens = batch * seqlen
    x_flat = x.reshape(n_tokens, hidden)
    vals_t, idx_t = pl.pallas_call(
        _router_body,
        in_specs=[
            pl.BlockSpec(memory_space=pl.ANY),
            pl.BlockSpec(memory_space=pltpu.VMEM),
        ],
        out_specs=[
            pl.BlockSpec(memory_space=pltpu.VMEM),
            pl.BlockSpec(memory_space=pltpu.VMEM),
        ],
        out_shape=[
            jax.ShapeDtypeStruct((TOPK, n_tokens), jnp.float32),
            jax.ShapeDtypeStruct((TOPK, n_tokens), jnp.int32),
        ],
        scratch_shapes=[
            pltpu.VMEM((NTBUF, TILE, hidden), jnp.float32),
            pltpu.SemaphoreType.DMA((NTBUF, CPT)),
        ],
    )(x_flat, W_router)
    return (vals_t.T, idx_t.T)
